# baseline (speedup 1.0000x reference)
import functools

import jax
import jax.numpy as jnp
from jax import lax
from jax.experimental import pallas as pl
from jax.experimental.pallas import tpu as pltpu
from jax.experimental.pallas import tpu_sc as plsc

B = 16384
K = 64
L = 16
NC = 2
NS = 16
NW = NC * NS
BPW = B // NW
CHUNK = 128
NCHUNK = BPW // CHUNK


def _sc_partials(sampleU, sampleI, sampleJ, betaI, gammaU, gammaI):
    mesh = plsc.VectorSubcoreMesh(core_axis_name="c", subcore_axis_name="s")

    @functools.partial(
        pl.kernel,
        out_type=(
            jax.ShapeDtypeStruct((B, L), jnp.float32),
            jax.ShapeDtypeStruct((B,), jnp.float32),
        ),
        mesh=mesh,
        compiler_params=pltpu.CompilerParams(use_tc_tiling_on_sc=False),
        scratch_types=[
            pltpu.VMEM((CHUNK,), jnp.int32),
            pltpu.VMEM((CHUNK,), jnp.int32),
            pltpu.VMEM((CHUNK,), jnp.int32),
            pltpu.VMEM((CHUNK, K), jnp.float32),
            pltpu.VMEM((CHUNK, K), jnp.float32),
            pltpu.VMEM((CHUNK, K), jnp.float32),
            pltpu.VMEM((CHUNK,), jnp.float32),
            pltpu.VMEM((CHUNK,), jnp.float32),
            pltpu.VMEM((CHUNK, L), jnp.float32),
            pltpu.VMEM((CHUNK,), jnp.float32),
            pltpu.SemaphoreType.DMA,
        ],
    )
    def sc_kernel(sU, sI, sJ, bI_hbm, gU_hbm, gI_hbm, part_hbm, bd_hbm,
                  idxU, idxI, idxJ, gU, gI, gJ, bIv, bJv, partv, bdv, sem):
        wid = lax.axis_index("s") * NC + lax.axis_index("c")
        base = wid * BPW

        def chunk_body(ci, carry):
            cbase = base + ci * CHUNK
            pltpu.sync_copy(sU.at[pl.ds(cbase, CHUNK)], idxU)
            pltpu.sync_copy(sI.at[pl.ds(cbase, CHUNK)], idxI)
            pltpu.sync_copy(sJ.at[pl.ds(cbase, CHUNK)], idxJ)
            cp1 = pltpu.async_copy(gU_hbm.at[idxU], gU, sem)
            cp2 = pltpu.async_copy(gI_hbm.at[idxI], gI, sem)
            cp3 = pltpu.async_copy(gI_hbm.at[idxJ], gJ, sem)
            cp4 = pltpu.async_copy(bI_hbm.at[idxI], bIv, sem)
            cp5 = pltpu.async_copy(bI_hbm.at[idxJ], bJv, sem)
            cp1.wait()
            cp2.wait()
            cp3.wait()
            cp4.wait()
            cp5.wait()

            def samp_body(s, c2):
                t = jnp.zeros((L,), jnp.float32)
                for q in range(K // L):
                    gu = gU[s, pl.ds(q * L, L)]
                    gi = gI[s, pl.ds(q * L, L)]
                    gj = gJ[s, pl.ds(q * L, L)]
                    t = t + gu * (gi - gj)
                partv[s, :] = t
                return c2

            lax.fori_loop(0, CHUNK, samp_body, 0, unroll=4)

            for g in range(CHUNK // L):
                bdv[pl.ds(g * L, L)] = bIv[pl.ds(g * L, L)] - bJv[pl.ds(g * L, L)]

            pltpu.sync_copy(partv, part_hbm.at[pl.ds(cbase, CHUNK), :])
            pltpu.sync_copy(bdv, bd_hbm.at[pl.ds(cbase, CHUNK)])
            return carry

        lax.fori_loop(0, NCHUNK, chunk_body, 0)

    return sc_kernel(sampleU, sampleI, sampleJ, betaI, gammaU, gammaI)


def _tc_loss(part, bdiff):
    def body(part_ref, bd_ref, out_ref):
        x = jnp.sum(part_ref[...], axis=1, keepdims=True) + bd_ref[...]
        loss = -jnp.mean(jnp.log(jax.nn.sigmoid(x)))
        out_ref[...] = loss.reshape(1, 1)

    out = pl.pallas_call(
        body,
        out_shape=jax.ShapeDtypeStruct((1, 1), jnp.float32),
    )(part, bdiff.reshape(B, 1))
    return out[0, 0]


def kernel(sampleU, sampleI, sampleJ, betaI, gammaU, gammaI):
    part, bdiff = _sc_partials(sampleU, sampleI, sampleJ, betaI, gammaU, gammaI)
    return _tc_loss(part, bdiff)


# speedup vs baseline: 1.0083x; 1.0083x over previous
import functools

import jax
import jax.numpy as jnp
from jax import lax
from jax.experimental import pallas as pl
from jax.experimental.pallas import tpu as pltpu
from jax.experimental.pallas import tpu_sc as plsc

B = 16384
K = 64
L = 16
NC = 2
NS = 16
NW = NC * NS
BPW = B // NW
CHUNK = 128
NCHUNK = BPW // CHUNK
NHALF = 1000000 // 2


def _sc_diffs(sampleU, sampleI, sampleJ, betaI, gammaU2, gammaI2):
    mesh = plsc.VectorSubcoreMesh(core_axis_name="c", subcore_axis_name="s")

    @functools.partial(
        pl.kernel,
        out_type=jax.ShapeDtypeStruct((B,), jnp.float32),
        mesh=mesh,
        compiler_params=pltpu.CompilerParams(
            use_tc_tiling_on_sc=True, needs_layout_passes=False),
        scratch_types=[
            pltpu.VMEM((CHUNK,), jnp.int32),
            pltpu.VMEM((CHUNK,), jnp.int32),
            pltpu.VMEM((CHUNK,), jnp.int32),
            pltpu.VMEM((CHUNK,), jnp.int32),
            pltpu.VMEM((CHUNK,), jnp.int32),
            pltpu.VMEM((CHUNK,), jnp.int32),
            pltpu.VMEM((CHUNK, 128), jnp.float32),
            pltpu.VMEM((CHUNK, 128), jnp.float32),
            pltpu.VMEM((CHUNK, 128), jnp.float32),
            pltpu.VMEM((CHUNK,), jnp.float32),
            pltpu.VMEM((CHUNK,), jnp.float32),
            pltpu.VMEM((CHUNK,), jnp.float32),
            pltpu.SemaphoreType.DMA,
        ],
    )
    def sc_kernel(sU, sI, sJ, bI_hbm, gU_hbm, gI_hbm, diff_hbm,
                  idxU, blkU, idxI, blkI, idxJ, blkJ,
                  gU, gI, gJ, bIv, bJv, dv, sem):
        wid = lax.axis_index("s") * NC + lax.axis_index("c")
        base = wid * BPW
        lanes = jnp.arange(L, dtype=jnp.int32)

        def chunk_body(ci, carry):
            cbase = base + ci * CHUNK
            pltpu.sync_copy(sU.at[pl.ds(cbase, CHUNK)], idxU)
            pltpu.sync_copy(sI.at[pl.ds(cbase, CHUNK)], idxI)
            pltpu.sync_copy(sJ.at[pl.ds(cbase, CHUNK)], idxJ)
            for g in range(CHUNK // L):
                sl = pl.ds(g * L, L)
                blkU[sl] = lax.shift_right_logical(idxU[sl], 1)
                blkI[sl] = lax.shift_right_logical(idxI[sl], 1)
                blkJ[sl] = lax.shift_right_logical(idxJ[sl], 1)
            cp1 = pltpu.async_copy(gU_hbm.at[blkU], gU, sem)
            cp2 = pltpu.async_copy(gI_hbm.at[blkI], gI, sem)
            cp3 = pltpu.async_copy(gI_hbm.at[blkJ], gJ, sem)
            cp4 = pltpu.async_copy(bI_hbm.at[idxI], bIv, sem)
            cp5 = pltpu.async_copy(bI_hbm.at[idxJ], bJv, sem)
            cp1.wait()
            cp2.wait()
            cp3.wait()
            cp4.wait()
            cp5.wait()

            for g in range(CHUNK // L):
                sl = pl.ds(g * L, L)
                svec = jnp.full((L,), g * L, jnp.int32) + lanes
                cu = lax.shift_left(lax.bitwise_and(idxU[sl], 1), 6)
                ci_ = lax.shift_left(lax.bitwise_and(idxI[sl], 1), 6)
                cj = lax.shift_left(lax.bitwise_and(idxJ[sl], 1), 6)
                acc = bIv[sl] - bJv[sl]
                for k in range(K):
                    kv = lax.bitwise_and(lanes + k, K - 1)
                    gu = plsc.load_gather(gU, [svec, cu + kv])
                    gi = plsc.load_gather(gI, [svec, ci_ + kv])
                    gj = plsc.load_gather(gJ, [svec, cj + kv])
                    acc = acc + gu * (gi - gj)
                dv[sl] = acc

            pltpu.sync_copy(dv, diff_hbm.at[pl.ds(cbase, CHUNK)])
            return carry

        lax.fori_loop(0, NCHUNK, chunk_body, 0)

    return sc_kernel(sampleU, sampleI, sampleJ, betaI, gammaU2, gammaI2)


def _tc_loss(diffs):
    def body(d_ref, out_ref):
        loss = -jnp.mean(jnp.log(jax.nn.sigmoid(d_ref[...])))
        out_ref[...] = loss.reshape(1, 1)

    out = pl.pallas_call(
        body,
        out_shape=jax.ShapeDtypeStruct((1, 1), jnp.float32),
    )(diffs.reshape(B // 128, 128))
    return out[0, 0]


def kernel(sampleU, sampleI, sampleJ, betaI, gammaU, gammaI):
    gU2 = gammaU.reshape(NHALF, 128)
    gI2 = gammaI.reshape(NHALF, 128)
    diffs = _sc_diffs(sampleU, sampleI, sampleJ, betaI, gU2, gI2)
    return _tc_loss(diffs)


# speedup vs baseline: 2.0750x; 2.0578x over previous
import functools

import jax
import jax.numpy as jnp
from jax import lax
from jax.experimental import pallas as pl
from jax.experimental.pallas import tpu as pltpu
from jax.experimental.pallas import tpu_sc as plsc

B = 16384
K = 64
L = 16
NC = 2
NS = 16
NW = NC * NS
BPW = B // NW
CHUNK = 32
NCHUNK = BPW // CHUNK
NBLK = 1000000 // 8


def _sc_diffs(sampleU, sampleI, sampleJ, betaI, gammaU3, gammaI3):
    mesh = plsc.VectorSubcoreMesh(core_axis_name="c", subcore_axis_name="s")

    @functools.partial(
        pl.kernel,
        out_type=jax.ShapeDtypeStruct((B,), jnp.float32),
        mesh=mesh,
        compiler_params=pltpu.CompilerParams(
            use_tc_tiling_on_sc=True, needs_layout_passes=False),
        scratch_types=[
            pltpu.VMEM((CHUNK,), jnp.int32),
            pltpu.VMEM((CHUNK,), jnp.int32),
            pltpu.VMEM((CHUNK,), jnp.int32),
            pltpu.VMEM((CHUNK, 8, K), jnp.float32),
            pltpu.VMEM((CHUNK, 8, K), jnp.float32),
            pltpu.VMEM((CHUNK, 8, K), jnp.float32),
            pltpu.VMEM((CHUNK,), jnp.float32),
            pltpu.VMEM((CHUNK,), jnp.float32),
            pltpu.VMEM((CHUNK,), jnp.float32),
            pltpu.SemaphoreType.DMA,
            pltpu.SemaphoreType.DMA,
        ],
    )
    def sc_kernel(sU, sI, sJ, bI_hbm, gU_hbm, gI_hbm, diff_hbm,
                  idxU, idxI, idxJ,
                  gU, gI, gJ, bIv, bJv, dv, sem, sem2):
        wid = lax.axis_index("s") * NC + lax.axis_index("c")
        base = wid * BPW
        lanes = jnp.arange(L, dtype=jnp.int32)

        def chunk_body(ci, carry):
            cbase = base + ci * CHUNK
            pltpu.sync_copy(sU.at[pl.ds(cbase, CHUNK)], idxU)
            pltpu.sync_copy(sI.at[pl.ds(cbase, CHUNK)], idxI)
            pltpu.sync_copy(sJ.at[pl.ds(cbase, CHUNK)], idxJ)
            cp4 = pltpu.async_copy(bI_hbm.at[idxI], bIv, sem2)
            cp5 = pltpu.async_copy(bI_hbm.at[idxJ], bJv, sem2)

            for g in range(CHUNK // L):
                sl = pl.ds(g * L, L)
                vu = lax.shift_right_logical(idxU[sl], 3)
                vi = lax.shift_right_logical(idxI[sl], 3)
                vj = lax.shift_right_logical(idxJ[sl], 3)
                for l in range(L):
                    s = g * L + l
                    pltpu.async_copy(gU_hbm.at[vu[l]], gU.at[s], sem)
                    pltpu.async_copy(gI_hbm.at[vi[l]], gI.at[s], sem)
                    pltpu.async_copy(gI_hbm.at[vj[l]], gJ.at[s], sem)
            for s in range(CHUNK):
                pltpu.make_async_copy(gU_hbm.at[0], gU.at[s], sem).wait()
                pltpu.make_async_copy(gU_hbm.at[0], gI.at[s], sem).wait()
                pltpu.make_async_copy(gU_hbm.at[0], gJ.at[s], sem).wait()
            cp4.wait()
            cp5.wait()

            for g in range(CHUNK // L):
                sl = pl.ds(g * L, L)
                svec = jnp.full((L,), g * L, jnp.int32) + lanes
                ru = lax.bitwise_and(idxU[sl], 7)
                ri = lax.bitwise_and(idxI[sl], 7)
                rj = lax.bitwise_and(idxJ[sl], 7)
                acc = bIv[sl] - bJv[sl]
                for k in range(K):
                    kv = lax.bitwise_and(lanes + k, K - 1)
                    gu = plsc.load_gather(gU, [svec, ru, kv])
                    gi = plsc.load_gather(gI, [svec, ri, kv])
                    gj = plsc.load_gather(gJ, [svec, rj, kv])
                    acc = acc + gu * (gi - gj)
                dv[sl] = acc

            pltpu.sync_copy(dv, diff_hbm.at[pl.ds(cbase, CHUNK)])
            return carry

        lax.fori_loop(0, NCHUNK, chunk_body, 0)

    return sc_kernel(sampleU, sampleI, sampleJ, betaI, gammaU3, gammaI3)


def _tc_loss(diffs):
    def body(d_ref, out_ref):
        loss = -jnp.mean(jnp.log(jax.nn.sigmoid(d_ref[...])))
        out_ref[...] = loss.reshape(1, 1)

    out = pl.pallas_call(
        body,
        out_shape=jax.ShapeDtypeStruct((1, 1), jnp.float32),
    )(diffs.reshape(B // 128, 128))
    return out[0, 0]


def kernel(sampleU, sampleI, sampleJ, betaI, gammaU, gammaI):
    gU3 = gammaU.reshape(NBLK, 8, K)
    gI3 = gammaI.reshape(NBLK, 8, K)
    diffs = _sc_diffs(sampleU, sampleI, sampleJ, betaI, gU3, gI3)
    return _tc_loss(diffs)
